# baseline (speedup 1.0000x reference)
import functools

import jax
import jax.numpy as jnp
from jax.experimental import pallas as pl
from jax.experimental.pallas import tpu as pltpu

_ROWS, _COLS = 16384, 50
_NCH = 16
_CW = _ROWS // _NCH


def _copy_body(x_ref, o_ref, buf, *sems):
    in_sems, out_sems = sems[:_NCH], sems[_NCH:]
    ins = [
        pltpu.make_async_copy(
            x_ref.at[:, pl.ds(k * _CW, _CW)],
            buf.at[:, pl.ds(k * _CW, _CW)],
            in_sems[k],
        )
        for k in range(_NCH)
    ]
    outs = [
        pltpu.make_async_copy(
            buf.at[:, pl.ds(k * _CW, _CW)],
            o_ref.at[:, pl.ds(k * _CW, _CW)],
            out_sems[k],
        )
        for k in range(_NCH)
    ]
    for c in ins:
        c.start()
    for k in range(_NCH):
        ins[k].wait()
        outs[k].start()
    for c in outs:
        c.wait()


@functools.cache
def _make_copy_kernel():
    return pl.pallas_call(
        _copy_body,
        in_specs=[pl.BlockSpec(memory_space=pl.ANY)],
        out_specs=pl.BlockSpec(memory_space=pl.ANY),
        out_shape=jax.ShapeDtypeStruct((_COLS, _ROWS), jnp.float32),
        scratch_shapes=[pltpu.VMEM((_COLS, _ROWS), jnp.float32)]
        + [pltpu.SemaphoreType.DMA] * (2 * _NCH),
    )


def kernel(candidate_titles):
    xt = pltpu.with_memory_space_constraint(
        candidate_titles.T, pltpu.MemorySpace.HBM
    )
    return _make_copy_kernel()(xt).T


# speedup vs baseline: 1.0492x; 1.0492x over previous
import functools

import jax
import jax.numpy as jnp
from jax.experimental import pallas as pl
from jax.experimental.pallas import tpu as pltpu

_ROWS, _COLS = 16384, 50
_NCH = 4
_CW = _ROWS // _NCH


def _copy_body(x_ref, o_ref, buf, *sems):
    in_sems, out_sems = sems[:_NCH], sems[_NCH:]
    ins = [
        pltpu.make_async_copy(
            x_ref.at[:, pl.ds(k * _CW, _CW)],
            buf.at[:, pl.ds(k * _CW, _CW)],
            in_sems[k],
        )
        for k in range(_NCH)
    ]
    outs = [
        pltpu.make_async_copy(
            buf.at[:, pl.ds(k * _CW, _CW)],
            o_ref.at[:, pl.ds(k * _CW, _CW)],
            out_sems[k],
        )
        for k in range(_NCH)
    ]
    for c in ins:
        c.start()
    for k in range(_NCH):
        ins[k].wait()
        outs[k].start()
    for c in outs:
        c.wait()


@functools.cache
def _make_copy_kernel():
    return pl.pallas_call(
        _copy_body,
        in_specs=[pl.BlockSpec(memory_space=pl.ANY)],
        out_specs=pl.BlockSpec(memory_space=pl.ANY),
        out_shape=jax.ShapeDtypeStruct((_COLS, _ROWS), jnp.float32),
        scratch_shapes=[pltpu.VMEM((_COLS, _ROWS), jnp.float32)]
        + [pltpu.SemaphoreType.DMA] * (2 * _NCH),
    )


def kernel(candidate_titles):
    xt = pltpu.with_memory_space_constraint(
        candidate_titles.T, pltpu.MemorySpace.HBM
    )
    return _make_copy_kernel()(xt).T
